# baseline (speedup 1.0000x reference)
import jax
import jax.numpy as jnp
from jax import lax
from jax.experimental import pallas as pl
from jax.experimental.pallas import tpu as pltpu
from jax.experimental.pallas import tpu_sc as plsc

N = 100000
D = 128
LANES = 16
NC = 2
NS = 16
NW = NC * NS

CH = 128
NFULL = N // CH
TAIL = N - NFULL * CH
TAIL_WID = NFULL % NW
MAXCH = (NFULL + NW - 1) // NW
MINCH = NFULL // NW


def _body(ordering_hbm, noise_hbm, table_hbm, mean_hbm, lstd_hbm, out_hbm,
          idx_v, rows_v, noise_v, mean_v, lstd_v,
          gsem0, gsem1, nsem0, nsem1, isem0, isem1, osem0, osem1, msem):
    gsem = [gsem0, gsem1]
    nsem = [nsem0, nsem1]
    isem = [isem0, isem1]
    osem = [osem0, osem1]

    wid = lax.axis_index("s") * NC + lax.axis_index("c")
    n_w = (NFULL + NW - 1 - wid) // NW
    starts = [(wid + NW * c) * CH for c in range(MAXCH)]

    def exists(c):
        if c < MINCH:
            return True
        if c >= MAXCH:
            return False
        return c < n_w

    def run_if(cond, fn):
        if cond is True:
            fn()
        elif cond is not False:
            def wrapped():
                fn()
            pl.when(cond)(wrapped)

    pltpu.async_copy(mean_hbm, mean_v, msem)
    pltpu.async_copy(lstd_hbm, lstd_v, msem)

    def compute_rows(buf, nrows):
        @plsc.parallel_loop(0, nrows, 1, unroll=4)
        def row_body(r):
            for j in range(D // LANES):
                sl = pl.ds(j * LANES, LANES)
                rows_v[buf, r, sl] = (rows_v[buf, r, sl] + mean_vals[j]
                                      + scale_vals[j] * noise_v[buf, r, sl])

    def issue_fetch(c, buf):
        pltpu.async_copy(table_hbm.at[idx_v.at[buf]], rows_v.at[buf],
                         gsem[buf])
        pltpu.async_copy(noise_hbm.at[pl.ds(starts[c], CH)],
                         noise_v.at[buf], nsem[buf])

    def issue_idx(c, buf):
        pltpu.async_copy(ordering_hbm.at[pl.ds(starts[c], CH)],
                         idx_v.at[buf], isem[buf])

    def wait_fetch(buf):
        pltpu.make_async_copy(table_hbm.at[idx_v.at[buf]], rows_v.at[buf],
                              gsem[buf]).wait()
        pltpu.make_async_copy(noise_hbm.at[pl.ds(0, CH)], noise_v.at[buf],
                              nsem[buf]).wait()

    def wait_idx(buf):
        pltpu.make_async_copy(ordering_hbm.at[pl.ds(0, CH)], idx_v.at[buf],
                              isem[buf]).wait()

    def wait_out(buf):
        pltpu.make_async_copy(rows_v.at[buf], out_hbm.at[pl.ds(0, CH)],
                              osem[buf]).wait()

    pltpu.sync_copy(ordering_hbm.at[pl.ds(starts[0], CH)], idx_v.at[0])
    issue_fetch(0, 0)
    issue_idx(1, 1)
    pltpu.make_async_copy(mean_hbm, mean_v, msem).wait()
    pltpu.make_async_copy(lstd_hbm, lstd_v, msem).wait()
    mean_vals = [mean_v[pl.ds(j * LANES, LANES)] for j in range(D // LANES)]
    scale_vals = [jnp.exp(lstd_v[pl.ds(j * LANES, LANES)])
                  for j in range(D // LANES)]

    for kk in range(MAXCH):
        b = kk % 2
        o = 1 - b
        run_if(exists(kk + 1), lambda: wait_idx(o))
        if kk >= 1:
            run_if(exists(kk + 1), lambda: wait_out(o))
        run_if(exists(kk + 1), lambda: issue_fetch(kk + 1, o))
        run_if(exists(kk), lambda: wait_fetch(b))
        run_if(exists(kk + 2), lambda: issue_idx(kk + 2, b))

        def produce(kk=kk, b=b):
            compute_rows(b, CH)
            pltpu.async_copy(rows_v.at[b],
                             out_hbm.at[pl.ds(starts[kk], CH)], osem[b])
        run_if(exists(kk), produce)

    wait_out(0)
    wait_out(1)

    @pl.when(wid == TAIL_WID)
    def _tail():
        start = NFULL * CH
        pltpu.sync_copy(ordering_hbm.at[pl.ds(start, TAIL)],
                        idx_v.at[0, pl.ds(0, TAIL)])
        gather = pltpu.async_copy(table_hbm.at[idx_v.at[0, pl.ds(0, TAIL)]],
                                  rows_v.at[0, pl.ds(0, TAIL)], gsem[0])
        pltpu.sync_copy(noise_hbm.at[pl.ds(start, TAIL)],
                        noise_v.at[0, pl.ds(0, TAIL)])
        gather.wait()

        def row_body(r, _):
            for j in range(D // LANES):
                sl = pl.ds(j * LANES, LANES)
                rows_v[0, r, sl] = (rows_v[0, r, sl] + mean_vals[j]
                                    + scale_vals[j] * noise_v[0, r, sl])
            return None
        lax.fori_loop(0, TAIL, row_body, None)
        pltpu.sync_copy(rows_v.at[0, pl.ds(0, TAIL)],
                        out_hbm.at[pl.ds(start, TAIL)])


@jax.jit
def _run(ordering, noise, pos_embed, rand_mean, rand_log_std):
    mesh = plsc.VectorSubcoreMesh(core_axis_name="c", subcore_axis_name="s",
                                  num_cores=NC, num_subcores=NS)
    f = pl.kernel(
        _body,
        out_type=jax.ShapeDtypeStruct((N, D), jnp.float32),
        mesh=mesh,
        scratch_types=[
            pltpu.VMEM((2, CH), jnp.int32),
            pltpu.VMEM((2, CH, D), jnp.float32),
            pltpu.VMEM((2, CH, D), jnp.float32),
            pltpu.VMEM((D,), jnp.float32),
            pltpu.VMEM((D,), jnp.float32),
        ] + [pltpu.SemaphoreType.DMA] * 9,
    )
    return f(ordering, noise, pos_embed, rand_mean, rand_log_std)


def kernel(ordering, noise, pos_embed, rand_mean, rand_log_std):
    return _run(ordering.astype(jnp.int32), noise, pos_embed,
                rand_mean, rand_log_std)


# speedup vs baseline: 1.0807x; 1.0807x over previous
import jax
import jax.numpy as jnp
from jax import lax
from jax.experimental import pallas as pl
from jax.experimental.pallas import tpu as pltpu
from jax.experimental.pallas import tpu_sc as plsc

N = 100000
D = 128
LANES = 16
NC = 2
NS = 16
NW = NC * NS

CH = 128
NFULL = N // CH
TAIL = N - NFULL * CH
TAIL_WID = NFULL % NW
MAXCH = (NFULL + NW - 1) // NW
MINCH = NFULL // NW


def _body(ordering_hbm, noise_hbm, table_hbm, mean_hbm, lstd_hbm, out_hbm,
          idx_v, rows_v, noise_v, mean_v, lstd_v,
          gsem0, gsem1, nsem0, nsem1, isem0, isem1, osem0, osem1, msem):
    gsem = [gsem0, gsem1]
    nsem = [nsem0, nsem1]
    isem = [isem0, isem1]
    osem = [osem0, osem1]

    wid = lax.axis_index("s") * NC + lax.axis_index("c")
    n_w = (NFULL + NW - 1 - wid) // NW
    starts = [(wid + NW * c) * CH for c in range(MAXCH)]

    def exists(c):
        if c < MINCH:
            return True
        if c >= MAXCH:
            return False
        return c < n_w

    def run_if(cond, fn):
        if cond is True:
            fn()
        elif cond is not False:
            def wrapped():
                fn()
            pl.when(cond)(wrapped)

    pltpu.async_copy(mean_hbm, mean_v, msem)
    pltpu.async_copy(lstd_hbm, lstd_v, msem)

    def compute_rows(buf, nrows):
        def row_body(r, _):
            for j in range(D // LANES):
                sl = pl.ds(j * LANES, LANES)
                rows_v[buf, r, sl] = (rows_v[buf, r, sl] + mean_vals[j]
                                      + scale_vals[j] * noise_v[buf, r, sl])
            return None
        lax.fori_loop(0, nrows, row_body, None)

    def issue_fetch(c, buf):
        pltpu.async_copy(table_hbm.at[idx_v.at[buf]], rows_v.at[buf],
                         gsem[buf])
        pltpu.async_copy(noise_hbm.at[pl.ds(starts[c], CH)],
                         noise_v.at[buf], nsem[buf])

    def issue_idx(c, buf):
        pltpu.async_copy(ordering_hbm.at[pl.ds(starts[c], CH)],
                         idx_v.at[buf], isem[buf])

    def wait_fetch(buf):
        pltpu.make_async_copy(table_hbm.at[idx_v.at[buf]], rows_v.at[buf],
                              gsem[buf]).wait()
        pltpu.make_async_copy(noise_hbm.at[pl.ds(0, CH)], noise_v.at[buf],
                              nsem[buf]).wait()

    def wait_idx(buf):
        pltpu.make_async_copy(ordering_hbm.at[pl.ds(0, CH)], idx_v.at[buf],
                              isem[buf]).wait()

    def wait_out(buf):
        pltpu.make_async_copy(rows_v.at[buf], out_hbm.at[pl.ds(0, CH)],
                              osem[buf]).wait()

    pltpu.sync_copy(ordering_hbm.at[pl.ds(starts[0], CH)], idx_v.at[0])
    issue_fetch(0, 0)
    issue_idx(1, 1)
    pltpu.make_async_copy(mean_hbm, mean_v, msem).wait()
    pltpu.make_async_copy(lstd_hbm, lstd_v, msem).wait()
    mean_vals = [mean_v[pl.ds(j * LANES, LANES)] for j in range(D // LANES)]
    scale_vals = [jnp.exp(lstd_v[pl.ds(j * LANES, LANES)])
                  for j in range(D // LANES)]

    for kk in range(MAXCH):
        b = kk % 2
        o = 1 - b
        run_if(exists(kk + 1), lambda: wait_idx(o))
        if kk >= 1:
            run_if(exists(kk + 1), lambda: wait_out(o))
        run_if(exists(kk + 1), lambda: issue_fetch(kk + 1, o))
        run_if(exists(kk), lambda: wait_fetch(b))
        run_if(exists(kk + 2), lambda: issue_idx(kk + 2, b))

        def produce(kk=kk, b=b):
            compute_rows(b, CH)
            pltpu.async_copy(rows_v.at[b],
                             out_hbm.at[pl.ds(starts[kk], CH)], osem[b])
        run_if(exists(kk), produce)

    wait_out(0)
    wait_out(1)

    @pl.when(wid == TAIL_WID)
    def _tail():
        start = NFULL * CH
        pltpu.sync_copy(ordering_hbm.at[pl.ds(start, TAIL)],
                        idx_v.at[0, pl.ds(0, TAIL)])
        gather = pltpu.async_copy(table_hbm.at[idx_v.at[0, pl.ds(0, TAIL)]],
                                  rows_v.at[0, pl.ds(0, TAIL)], gsem[0])
        pltpu.sync_copy(noise_hbm.at[pl.ds(start, TAIL)],
                        noise_v.at[0, pl.ds(0, TAIL)])
        gather.wait()

        def row_body(r, _):
            for j in range(D // LANES):
                sl = pl.ds(j * LANES, LANES)
                rows_v[0, r, sl] = (rows_v[0, r, sl] + mean_vals[j]
                                    + scale_vals[j] * noise_v[0, r, sl])
            return None
        lax.fori_loop(0, TAIL, row_body, None)
        pltpu.sync_copy(rows_v.at[0, pl.ds(0, TAIL)],
                        out_hbm.at[pl.ds(start, TAIL)])


@jax.jit
def _run(ordering, noise, pos_embed, rand_mean, rand_log_std):
    mesh = plsc.VectorSubcoreMesh(core_axis_name="c", subcore_axis_name="s",
                                  num_cores=NC, num_subcores=NS)
    f = pl.kernel(
        _body,
        out_type=jax.ShapeDtypeStruct((N, D), jnp.float32),
        mesh=mesh,
        scratch_types=[
            pltpu.VMEM((2, CH), jnp.int32),
            pltpu.VMEM((2, CH, D), jnp.float32),
            pltpu.VMEM((2, CH, D), jnp.float32),
            pltpu.VMEM((D,), jnp.float32),
            pltpu.VMEM((D,), jnp.float32),
        ] + [pltpu.SemaphoreType.DMA] * 9,
    )
    return f(ordering, noise, pos_embed, rand_mean, rand_log_std)


def kernel(ordering, noise, pos_embed, rand_mean, rand_log_std):
    return _run(ordering.astype(jnp.int32), noise, pos_embed,
                rand_mean, rand_log_std)
